# baseline (speedup 1.0000x reference)
import functools

import jax
import jax.numpy as jnp
from jax import lax
from jax.experimental import pallas as pl
from jax.experimental.pallas import tpu as pltpu
from jax.experimental.pallas import tpu_sc as plsc

_NC, _NS = 2, 16
_CH = 64
_LANES = 16


def _build_sc_kernel(B, LV, C, L, Lp):
    TROWS = LV + 1
    n_chunks = L // _CH
    per_half = (n_chunks + 1) // 2
    mesh = plsc.VectorSubcoreMesh(core_axis_name="c", subcore_axis_name="s")

    @functools.partial(
        pl.kernel,
        mesh=mesh,
        out_type=jax.ShapeDtypeStruct((B, L, C), jnp.float32),
        scratch_types=[
            pltpu.VMEM((Lp,), jnp.int32),
            pltpu.VMEM((Lp,), jnp.int32),
            pltpu.VMEM((_CH, C), jnp.float32),
            pltpu.VMEM((_CH, C), jnp.float32),
            pltpu.VMEM((8, C), jnp.float32),
            pltpu.VMEM((8, C), jnp.float32),
            pltpu.SemaphoreType.DMA,
            pltpu.SemaphoreType.DMA,
        ],
    )
    def sc_kernel(x_hbm, pol_hbm, pe_hbm, out_hbm, pol_v, idx_v, pe_v, g_v,
                  pe_t, g_t, sem1, sem2):
        wid = lax.axis_index("s") * _NC + lax.axis_index("c")
        b = wid // 2
        h = wid % 2
        bbase = b * TROWS

        pltpu.sync_copy(pol_hbm.at[b], pol_v)

        lanes = lax.iota(jnp.int32, _LANES)

        def _take(v, i):
            return v.at[i].get(mode="promise_in_bounds")

        def scan_body(i, carry):
            ch = pol_v[pl.ds(i * _LANES, _LANES)]
            cs = ch
            for d in (1, 2, 4, 8):
                shifted = _take(cs, jnp.maximum(lanes - d, 0))
                cs = cs + jnp.where(lanes >= d, shifted, 0)
            excl = cs - ch + carry
            idx_v[pl.ds(i * _LANES, _LANES)] = bbase + LV + ch * (excl - LV)
            return carry + _take(cs, jnp.full((_LANES,), _LANES - 1, jnp.int32))

        lax.fori_loop(0, Lp // _LANES, scan_body,
                      jnp.zeros((_LANES,), jnp.int32))

        def do_chunk(start, pe_b, g_b, out_rows):
            nrows = pe_b.shape[0]
            cp1 = pltpu.async_copy(pe_hbm.at[pl.ds(start, nrows)], pe_b, sem1)
            cp2 = pltpu.async_copy(x_hbm.at[idx_v.at[pl.ds(start, nrows)]],
                                   g_b, sem2)
            cp1.wait()
            cp2.wait()

            def row_body(r, _):
                for cc in range(C // _LANES):
                    sl = pl.ds(cc * _LANES, _LANES)
                    pe_b[r, sl] = pe_b[r, sl] + g_b[r, sl]
                return 0

            lax.fori_loop(0, out_rows, row_body, 0)
            src = pe_b if out_rows == nrows else pe_b.at[pl.ds(0, out_rows)]
            pltpu.sync_copy(src, out_hbm.at[b, pl.ds(start, out_rows)])

        for k in range(per_half):
            u = h * (n_chunks - per_half) + k
            do_chunk(u * _CH, pe_v, g_v, _CH)

        @pl.when(h == 1)
        def _():
            do_chunk(n_chunks * _CH, pe_t, g_t, 1)

    return sc_kernel


def kernel(x, policy, mask_token, pos_embed):
    B, LV, C = x.shape
    L = policy.shape[1]
    Lp = ((L + _LANES - 1) // _LANES) * _LANES

    polp = jnp.pad(policy.astype(jnp.int32), ((0, 0), (0, Lp - L)))
    x_ext = jnp.concatenate(
        [x, jnp.broadcast_to(mask_token, (B, 1, C)).astype(x.dtype)], axis=1
    ).reshape(B * (LV + 1), C)
    pe = jnp.pad(pos_embed.reshape(L, C), ((0, Lp - L), (0, 0)))

    sc = _build_sc_kernel(B, LV, C, L, Lp)
    return sc(x_ext, polp, pe)
